# Initial kernel scaffold; baseline (speedup 1.0000x reference)
import functools

import jax
import jax.numpy as jnp
from jax import lax
from jax.experimental import pallas as pl
from jax.experimental.pallas import tpu as pltpu
from jax.experimental.pallas import tpu_sc as plsc

N = 10000
E = 320000
NC = 2
NS = 16
EPT = E // (NC * NS)
K = 80
CH = EPT // K
RPT = N // NS
ZR = 25


def _sc_edge_scatter(D):
    mesh = plsc.VectorSubcoreMesh(core_axis_name="c", subcore_axis_name="s")

    def body(h, src, dst, out, sidx, didx, rows, zbuf, acc, sem):
        cid = lax.axis_index("c")
        sid = lax.axis_index("s")

        def zrow(r, _):
            for j in range(D // 16):
                zbuf[r, pl.ds(j * 16, 16)] = jnp.zeros((16,), jnp.float32)
            return 0
        lax.fori_loop(0, ZR, zrow, 0)

        def zacc(j, _):
            pltpu.sync_copy(zbuf, acc.at[pl.ds(sid * RPT + j * ZR, ZR)])
            return 0
        lax.fori_loop(0, RPT // ZR, zacc, 0)

        pltpu.sync_copy(src.at[cid, sid], sidx)
        pltpu.sync_copy(dst.at[cid, sid], didx)

        plsc.subcore_barrier()

        def step(i, _):
            pltpu.async_copy(h.at[sidx.at[i]], rows, sem).wait()
            pltpu.sync_copy(rows, acc.at[didx.at[i]], add=True)
            return 0
        lax.fori_loop(0, CH, step, 0)

        plsc.subcore_barrier()
        pltpu.sync_copy(acc.at[pl.ds(sid * RPT, RPT)],
                        out.at[cid, pl.ds(sid * RPT, RPT)])

    return pl.kernel(
        body,
        out_type=jax.ShapeDtypeStruct((NC, N, D), jnp.float32),
        mesh=mesh,
        scratch_types=[
            pltpu.VMEM((CH, K), jnp.int32),
            pltpu.VMEM((CH, K), jnp.int32),
            pltpu.VMEM((K, D), jnp.float32),
            pltpu.VMEM((ZR, D), jnp.float32),
            pltpu.VMEM_SHARED((N, D), jnp.float32),
            pltpu.SemaphoreType.DMA,
        ],
    )


def _mm_body(x_ref, w_ref, o_ref):
    o_ref[...] = jnp.dot(x_ref[...], w_ref[...],
                         preferred_element_type=jnp.float32)


def _relu_mm_body(p_ref, w_ref, o_ref):
    r = jnp.maximum(p_ref[0] + p_ref[1], 0.0)
    o_ref[...] = jnp.dot(r, w_ref[...], preferred_element_type=jnp.float32)


def _add_body(q_ref, o_ref):
    o_ref[...] = q_ref[0] + q_ref[1]


@functools.lru_cache(maxsize=None)
def _layers():
    return _sc_edge_scatter(128), _sc_edge_scatter(64)


def kernel(x, adj, W1, W2):
    src = adj[0].astype(jnp.int32).reshape(NC, NS, CH, K)
    dst = adj[1].astype(jnp.int32).reshape(NC, NS, CH, K)
    sc1, sc2 = _layers()

    h1 = pl.pallas_call(
        _mm_body,
        out_shape=jax.ShapeDtypeStruct((N, 128), jnp.float32),
    )(x, W1)
    p1 = sc1(h1, src, dst)
    h2 = pl.pallas_call(
        _relu_mm_body,
        out_shape=jax.ShapeDtypeStruct((N, 64), jnp.float32),
    )(p1, W2)
    p2 = sc2(h2, src, dst)
    out = pl.pallas_call(
        _add_body,
        out_shape=jax.ShapeDtypeStruct((N, 64), jnp.float32),
    )(p2)
    return out


# speedup vs baseline: 8.0358x; 8.0358x over previous
import functools

import jax
import jax.numpy as jnp
from jax import lax
from jax.experimental import pallas as pl
from jax.experimental.pallas import tpu as pltpu
from jax.experimental.pallas import tpu_sc as plsc

N = 10000
E = 320000
NC = 2
NS = 16
EPT = E // (NC * NS)
K = 80
CH = EPT // K
RPT = 624
REM = N - NS * RPT
ZR = 48


def _sc_edge_scatter(D):
    mesh = plsc.VectorSubcoreMesh(core_axis_name="c", subcore_axis_name="s")

    def body(h, src, dst, out, sidx, didx, rows, zbuf, acc, sem):
        cid = lax.axis_index("c")
        sid = lax.axis_index("s")

        def zrow(r, _):
            for j in range(D // 16):
                zbuf[r, pl.ds(j * 16, 16)] = jnp.zeros((16,), jnp.float32)
            return 0
        lax.fori_loop(0, ZR, zrow, 0)

        def zacc(j, _):
            pltpu.sync_copy(zbuf, acc.at[pl.ds(sid * RPT + j * ZR, ZR)])
            return 0
        lax.fori_loop(0, RPT // ZR, zacc, 0)

        @pl.when(sid == NS - 1)
        def _():
            pltpu.sync_copy(zbuf.at[pl.ds(0, REM)],
                            acc.at[pl.ds(NS * RPT, REM)])

        pltpu.sync_copy(src.at[cid, sid], sidx)
        pltpu.sync_copy(dst.at[cid, sid], didx)

        plsc.subcore_barrier()

        def step(i, _):
            pltpu.async_copy(h.at[sidx.at[i]], rows, sem).wait()
            pltpu.sync_copy(rows, acc.at[didx.at[i]], add=True)
            return 0
        lax.fori_loop(0, CH, step, 0)

        plsc.subcore_barrier()
        pltpu.sync_copy(acc.at[pl.ds(sid * RPT, RPT)],
                        out.at[cid, pl.ds(sid * RPT, RPT)])

        @pl.when(sid == NS - 1)
        def _():
            pltpu.sync_copy(acc.at[pl.ds(NS * RPT, REM)],
                            out.at[cid, pl.ds(NS * RPT, REM)])

    return pl.kernel(
        body,
        out_type=jax.ShapeDtypeStruct((NC, N, D), jnp.float32),
        mesh=mesh,
        scratch_types=[
            pltpu.VMEM((CH, K), jnp.int32),
            pltpu.VMEM((CH, K), jnp.int32),
            pltpu.VMEM((K, D), jnp.float32),
            pltpu.VMEM((ZR, D), jnp.float32),
            pltpu.VMEM_SHARED((N, D), jnp.float32),
            pltpu.SemaphoreType.DMA,
        ],
        compiler_params=pltpu.CompilerParams(use_tc_tiling_on_sc=False),
    )


def _mm_body(x_ref, w_ref, o_ref):
    o_ref[...] = jnp.dot(x_ref[...], w_ref[...],
                         preferred_element_type=jnp.float32)


def _relu_mm_body(p_ref, w_ref, o_ref):
    r = jnp.maximum(p_ref[0] + p_ref[1], 0.0)
    o_ref[...] = jnp.dot(r, w_ref[...], preferred_element_type=jnp.float32)


def _add_body(q_ref, o_ref):
    o_ref[...] = q_ref[0] + q_ref[1]


@functools.lru_cache(maxsize=None)
def _layers():
    return _sc_edge_scatter(128), _sc_edge_scatter(64)


def kernel(x, adj, W1, W2):
    src = adj[0].astype(jnp.int32).reshape(NC, NS, CH, K)
    dst = adj[1].astype(jnp.int32).reshape(NC, NS, CH, K)
    sc1, sc2 = _layers()

    h1 = pl.pallas_call(
        _mm_body,
        out_shape=jax.ShapeDtypeStruct((N, 128), jnp.float32),
    )(x, W1)
    p1 = sc1(h1, src, dst)
    h2 = pl.pallas_call(
        _relu_mm_body,
        out_shape=jax.ShapeDtypeStruct((N, 64), jnp.float32),
    )(p1, W2)
    p2 = sc2(h2, src, dst)
    out = pl.pallas_call(
        _add_body,
        out_shape=jax.ShapeDtypeStruct((N, 64), jnp.float32),
    )(p2)
    return out
